# baseline (speedup 1.0000x reference)
import jax
import jax.numpy as jnp
from jax.experimental import pallas as pl

_NP = 98
_HID = 128
_NL = 4
_INCH = 256
_HW = 32 * 32


def _fused_kernel(x_ref, cw_ref, cb_ref, win_ref, bin_ref,
                  wq_ref, wk_ref, wg_ref, bg_ref, wout_ref, bout_ref,
                  out_ref):
    xb = x_ref[0]
    acc = jnp.zeros((_NP, _HID), jnp.float32)
    for j in range(4):
        hj = jnp.dot(cw_ref[j], xb, preferred_element_type=jnp.float32)
        hj = jnp.maximum(hj + cb_ref[j], 0.0)
        acc = acc + jnp.dot(hj, win_ref[j], preferred_element_type=jnp.float32)
    x = jnp.maximum(acc + bin_ref[...], 0.0)
    scale = 1.0 / jnp.sqrt(jnp.float32(_HID))
    for l in range(_NL):
        q = jnp.dot(x, wq_ref[l], preferred_element_type=jnp.float32)
        k = jnp.dot(x, wk_ref[l], preferred_element_type=jnp.float32)
        logits = jax.lax.dot_general(
            q, k, (((1,), (1,)), ((), ())),
            preferred_element_type=jnp.float32) * scale
        a = jax.nn.softmax(logits, axis=-1)
        g = jnp.dot(x, wg_ref[l], preferred_element_type=jnp.float32)
        msg = jnp.dot(a, g, preferred_element_type=jnp.float32) + bg_ref[l]
        x = jnp.maximum(msg + x, 0.0)
    out_ref[0] = (jnp.dot(x, wout_ref[...], preferred_element_type=jnp.float32)
                  + bout_ref[...])


def kernel(x_dict, conv_w, conv_b, W_in, b_in, Wq, Wk, Wg, bg, W_out, b_out):
    b = x_dict.shape[0]
    xr = x_dict.reshape(b, _INCH, _HW)
    cw_r = conv_w.reshape(_NP, 4, _INCH).transpose(1, 0, 2)
    cb_r = conv_b.reshape(_NP, 4).T.reshape(4, _NP, 1)
    win_r = W_in.reshape(4, _HW, _HID)
    bin_r = b_in.reshape(1, _HID)
    bg_r = bg.reshape(_NL, 1, _HID)
    wout_p = jnp.zeros((_HID, _HID), jnp.float32).at[:, :2].set(W_out)
    bout_p = jnp.zeros((1, _HID), jnp.float32).at[0, :2].set(b_out)

    out = pl.pallas_call(
        _fused_kernel,
        grid=(b,),
        in_specs=[
            pl.BlockSpec((1, _INCH, _HW), lambda i: (i, 0, 0)),
            pl.BlockSpec((4, _NP, _INCH), lambda i: (0, 0, 0)),
            pl.BlockSpec((4, _NP, 1), lambda i: (0, 0, 0)),
            pl.BlockSpec((4, _HW, _HID), lambda i: (0, 0, 0)),
            pl.BlockSpec((1, _HID), lambda i: (0, 0)),
            pl.BlockSpec((_NL, _HID, _HID), lambda i: (0, 0, 0)),
            pl.BlockSpec((_NL, _HID, _HID), lambda i: (0, 0, 0)),
            pl.BlockSpec((_NL, _HID, _HID), lambda i: (0, 0, 0)),
            pl.BlockSpec((_NL, 1, _HID), lambda i: (0, 0, 0)),
            pl.BlockSpec((_HID, _HID), lambda i: (0, 0)),
            pl.BlockSpec((1, _HID), lambda i: (0, 0)),
        ],
        out_specs=pl.BlockSpec((1, _NP, _HID), lambda i: (i, 0, 0)),
        out_shape=jax.ShapeDtypeStruct((b, _NP, _HID), jnp.float32),
    )(xr, cw_r, cb_r, win_r, bin_r, Wq, Wk, Wg, bg_r, wout_p, bout_p)
    return out[:, :, :2].reshape(b, -1)


# speedup vs baseline: 1.0305x; 1.0305x over previous
import jax
import jax.numpy as jnp
from jax.experimental import pallas as pl
from jax.experimental.pallas import tpu as pltpu

_NP = 98
_HID = 128
_NL = 4
_INCH = 256
_HW = 32 * 32
_NB = 4


def _fused_kernel(x_ref, cw_ref, cb_ref, win_ref, bin_ref,
                  wq_ref, wk_ref, wg_ref, bg_ref, wout_ref, bout_ref,
                  out_ref):
    scale = 1.0 / jnp.sqrt(jnp.float32(_HID))
    for s in range(_NB):
        xb = x_ref[s]
        acc = jnp.zeros((_NP, _HID), jnp.float32)
        for j in range(4):
            hj = jnp.dot(cw_ref[j], xb, preferred_element_type=jnp.float32)
            hj = jnp.maximum(hj + cb_ref[j], 0.0)
            acc = acc + jnp.dot(hj, win_ref[j],
                                preferred_element_type=jnp.float32)
        x = jnp.maximum(acc + bin_ref[...], 0.0)
        for l in range(_NL):
            q = jnp.dot(x, wq_ref[l], preferred_element_type=jnp.float32)
            k = jnp.dot(x, wk_ref[l], preferred_element_type=jnp.float32)
            logits = jax.lax.dot_general(
                q, k, (((1,), (1,)), ((), ())),
                preferred_element_type=jnp.float32) * scale
            a = jax.nn.softmax(logits, axis=-1)
            g = jnp.dot(x, wg_ref[l], preferred_element_type=jnp.float32)
            msg = jnp.dot(a, g, preferred_element_type=jnp.float32) + bg_ref[l]
            x = jnp.maximum(msg + x, 0.0)
        out_ref[s] = (jnp.dot(x, wout_ref[...],
                              preferred_element_type=jnp.float32)
                      + bout_ref[...])


def kernel(x_dict, conv_w, conv_b, W_in, b_in, Wq, Wk, Wg, bg, W_out, b_out):
    b = x_dict.shape[0]
    xr = x_dict.reshape(b, _INCH, _HW)
    cw_r = conv_w.reshape(_NP, 4, _INCH).transpose(1, 0, 2)
    cb_r = conv_b.reshape(_NP, 4).T.reshape(4, _NP, 1)
    win_r = W_in.reshape(4, _HW, _HID)
    bin_r = b_in.reshape(1, _HID)
    bg_r = bg.reshape(_NL, 1, _HID)
    wout_p = jnp.zeros((_HID, _HID), jnp.float32).at[:, :2].set(W_out)
    bout_p = jnp.zeros((1, _HID), jnp.float32).at[0, :2].set(b_out)

    out = pl.pallas_call(
        _fused_kernel,
        grid=(b // _NB,),
        compiler_params=pltpu.CompilerParams(
            dimension_semantics=("parallel",)),
        in_specs=[
            pl.BlockSpec((_NB, _INCH, _HW), lambda i: (i, 0, 0)),
            pl.BlockSpec((4, _NP, _INCH), lambda i: (0, 0, 0)),
            pl.BlockSpec((4, _NP, 1), lambda i: (0, 0, 0)),
            pl.BlockSpec((4, _HW, _HID), lambda i: (0, 0, 0)),
            pl.BlockSpec((1, _HID), lambda i: (0, 0)),
            pl.BlockSpec((_NL, _HID, _HID), lambda i: (0, 0, 0)),
            pl.BlockSpec((_NL, _HID, _HID), lambda i: (0, 0, 0)),
            pl.BlockSpec((_NL, _HID, _HID), lambda i: (0, 0, 0)),
            pl.BlockSpec((_NL, 1, _HID), lambda i: (0, 0, 0)),
            pl.BlockSpec((_HID, _HID), lambda i: (0, 0)),
            pl.BlockSpec((1, _HID), lambda i: (0, 0)),
        ],
        out_specs=pl.BlockSpec((_NB, _NP, _HID), lambda i: (i, 0, 0)),
        out_shape=jax.ShapeDtypeStruct((b, _NP, _HID), jnp.float32),
    )(xr, cw_r, cb_r, win_r, bin_r, Wq, Wk, Wg, bg_r, wout_p, bout_p)
    return out[:, :, :2].reshape(b, -1)
